# baseline (speedup 1.0000x reference)
import functools

import jax
import jax.numpy as jnp
from jax import lax
from jax.experimental import pallas as pl
from jax.experimental.pallas import tpu as pltpu
from jax.experimental.pallas import tpu_sc as plsc

L = 5
N = 10000
E = 160000
D = 256
H = 512
EDGE_DIM = 7

DH = D // 2
NT = 16
EPT = E // NT
K = 80
NCHUNK = EPT // K
ROWS_PT = 640
ROWS_LAST = N - 15 * ROWS_PT


NB = 3
CPT = EPT // K


def _sc_msg_body(layer, h_hbm, src_hbm, dst_hbm, emb_hbm, out_hbm,
                 src0, src1, src2, dst0, dst1, dst2, rows0, rows1, rows2,
                 emb0, emb1, emb2, agg_sh,
                 g0, g1, g2, m0, m1, m2, s0, s1, s2, d0, d1, d2, c0, c1, c2):
    c = lax.axis_index("c")
    s = lax.axis_index("s")
    rows = [rows0, rows1, rows2]
    embs = [emb0, emb1, emb2]
    srcs = [src0, src1, src2]
    dsts = [dst0, dst1, dst2]
    gsem = [g0, g1, g2]
    msem = [m0, m1, m2]
    ssem = [s0, s1, s2]
    dsem = [d0, d1, d2]
    csem = [c0, c1, c2]

    e0 = s * EPT
    erow0 = (2 * layer * E + c * E + e0) // K

    def _zero_row(e, carry):
        for j in range(DH // 16):
            rows0[e, pl.ds(j * 16, 16)] = jnp.zeros((16,), jnp.float32)
        return carry
    lax.fori_loop(0, K, _zero_row, 0)

    r0 = s * ROWS_PT
    nz = jnp.where(s == NT - 1, ROWS_LAST // K, ROWS_PT // K)

    def _zero_agg(i, carry):
        pltpu.sync_copy(rows0, agg_sh.at[pl.ds(r0 + i * K, K)])
        return carry
    lax.fori_loop(0, nz, _zero_agg, 0)

    plsc.subcore_barrier()

    def _issue_src(i, b):
        return pltpu.async_copy(src_hbm.at[pl.ds(c * E + e0 + i * K, K)],
                                srcs[b], csem[b])

    def _wait_src(i, b):
        pltpu.make_async_copy(src_hbm.at[pl.ds(c * E + e0 + i * K, K)],
                              srcs[b], csem[b]).wait()

    def _issue_gather(b):
        return pltpu.async_copy(h_hbm.at[srcs[b]], rows[b], gsem[b])

    def _issue_emb(i, b):
        return pltpu.async_copy(
            emb_hbm.at[pl.ds((erow0 + i) * (K // 2), K // 2)],
            embs[b], msem[b])

    def _issue_dst(i, b):
        return pltpu.async_copy(dst_hbm.at[pl.ds(e0 + i * K, K)],
                                dsts[b], dsem[b])

    def _issue_scatter(b):
        return pltpu.async_copy(rows[b], agg_sh.at[dsts[b]], ssem[b],
                                add=True)

    def _wait_gather(b):
        pltpu.make_async_copy(h_hbm.at[srcs[b]], rows[b], gsem[b]).wait()

    def _wait_emb(i, b):
        pltpu.make_async_copy(
            emb_hbm.at[pl.ds((erow0 + i) * (K // 2), K // 2)],
            embs[b], msem[b]).wait()

    def _wait_dst(i, b):
        pltpu.make_async_copy(dst_hbm.at[pl.ds(e0 + i * K, K)],
                              dsts[b], dsem[b]).wait()

    def _wait_scatter(b):
        pltpu.make_async_copy(rows[b], agg_sh.at[dsts[b]], ssem[b]).wait()

    def _compute(b):
        rv = rows[b]
        ev = embs[b]

        def _edge(e2, cc):
            for u2 in range(2):
                e = e2 * 2 + u2
                for j in range(DH // 32):
                    u = ev[e2, pl.ds(u2 * 64 + j * 16, 16)]
                    lo = lax.bitcast_convert_type(u << 16, jnp.float32)
                    hi = lax.bitcast_convert_type(u & jnp.int32(-65536),
                                                  jnp.float32)
                    sl0 = pl.ds(j * 32, 16)
                    sl1 = pl.ds(j * 32 + 16, 16)
                    rv[e, sl0] = jnp.maximum(rv[e, sl0] + lo, 0.0)
                    rv[e, sl1] = jnp.maximum(rv[e, sl1] + hi, 0.0)
            return cc
        lax.fori_loop(0, K // 2, _edge, 0)

    _issue_src(0, 0)
    _issue_src(1, 1)
    _issue_src(2, 2)
    _issue_emb(0, 0)
    _issue_emb(1, 1)
    _issue_emb(2, 2)
    _issue_dst(0, 0)
    _issue_dst(1, 1)
    _wait_src(0, 0)
    _issue_gather(0)
    _wait_src(1, 1)
    _issue_gather(1)

    def _step(i, b, g=None):
        tail = g is None
        b2 = (b + 2) % NB
        _wait_gather(b)
        if tail:
            if i >= 1:
                _wait_scatter(b2)
        elif b == 0:
            @pl.when(g >= 1)
            def _():
                _wait_scatter(b2)
        else:
            _wait_scatter(b2)
        if (not tail) or i + 2 <= CPT - 1:
            _wait_src(i + 2, b2)
            _issue_gather(b2)
            _issue_dst(i + 2, b2)
        if (not tail) or i + 3 <= CPT - 1:
            _issue_src(i + 3, b)
        _wait_emb(i, b)
        _compute(b)
        _wait_dst(i, b)
        if (not tail) or i < CPT - 1:
            _issue_scatter(b)
        else:
            pltpu.sync_copy(rows[b], agg_sh.at[dsts[b]], add=True)
        if (not tail) or i + 3 <= CPT - 1:
            _issue_emb(i + 3, b)

    NMAIN = ((CPT - 5) // NB) * NB

    def _main(g, carry):
        for b in range(NB):
            _step(g * NB + b, b, g=g)
        return carry
    lax.fori_loop(0, NMAIN // NB, _main, 0)

    for i in range(NMAIN, CPT):
        _step(i, i % NB)

    plsc.subcore_barrier()

    def _wb(i, carry):
        pltpu.sync_copy(agg_sh.at[pl.ds(r0 + i * K, K)], rows0)
        pltpu.sync_copy(rows0, out_hbm.at[pl.ds(c * N + r0 + i * K, K)])
        return carry
    lax.fori_loop(0, nz, _wb, 0)


@functools.cache
def _sc_msg_fn(layer):
    return pl.kernel(
        functools.partial(_sc_msg_body, layer),
        out_type=jax.ShapeDtypeStruct((2 * N, DH), jnp.float32),
        mesh=plsc.VectorSubcoreMesh(core_axis_name="c", subcore_axis_name="s"),
        scratch_types=(
            [pltpu.VMEM((K,), jnp.int32) for _ in range(2 * NB)]
            + [pltpu.VMEM((K, DH), jnp.float32) for _ in range(NB)]
            + [pltpu.VMEM((K // 2, DH), jnp.int32) for _ in range(NB)]
            + [pltpu.VMEM_SHARED((N, DH), jnp.float32)]
            + [pltpu.SemaphoreType.DMA for _ in range(5 * NB)]
        ),
    )


def _edge_emb_kernel(attr_ref, Wt_ref, bt_ref, out_ref):
    emb = jnp.dot(attr_ref[...], Wt_ref[...], preferred_element_type=jnp.float32)
    emb = (emb + bt_ref[...]).astype(jnp.bfloat16)
    for l in range(L):
        out_ref[l, 0] = emb[:, l * D:l * D + DH]
        out_ref[l, 1] = emb[:, l * D + DH:(l + 1) * D]


def _mlp_kernel(eps_ref, h_ref, agg_ref, W1_ref, b1_ref, W2_ref, b2_ref,
                out_ref, *, relu_out):
    h = jnp.concatenate([h_ref[0], h_ref[1]], axis=1)
    a = jnp.concatenate([agg_ref[0], agg_ref[1]], axis=1)
    pre = eps_ref[0] * h + a
    mid = jnp.dot(pre, W1_ref[...], preferred_element_type=jnp.float32)
    mid = jnp.maximum(mid + b1_ref[...], 0.0)
    out = jnp.dot(mid, W2_ref[...], preferred_element_type=jnp.float32)
    out = out + b2_ref[...]
    if relu_out:
        out = jnp.maximum(out, 0.0)
    out_ref[0] = out[:, :DH]
    out_ref[1] = out[:, DH:]


_EEB = 1000
_NB = 1000


def _edge_emb_all(edge_attr, edgeWt, edgebt):
    return pl.pallas_call(
        _edge_emb_kernel,
        grid=(E // _EEB,),
        in_specs=[
            pl.BlockSpec((_EEB, EDGE_DIM), lambda e: (e, 0)),
            pl.BlockSpec((EDGE_DIM, L * D), lambda e: (0, 0)),
            pl.BlockSpec((1, L * D), lambda e: (0, 0)),
        ],
        out_specs=pl.BlockSpec((L, 2, _EEB, DH), lambda e: (0, 0, e, 0)),
        out_shape=jax.ShapeDtypeStruct((L, 2, E, DH), jnp.bfloat16),
    )(edge_attr, edgeWt, edgebt)


def _mlp(epsv, h_split, agg_split, W1f, b1f, W2f, b2f, relu_out):
    return pl.pallas_call(
        functools.partial(_mlp_kernel, relu_out=relu_out),
        grid=(N // _NB,),
        in_specs=[
            pl.BlockSpec(memory_space=pltpu.SMEM),
            pl.BlockSpec((2, _NB, DH), lambda i: (0, i, 0)),
            pl.BlockSpec((2, _NB, DH), lambda i: (0, i, 0)),
            pl.BlockSpec((D, H), lambda i: (0, 0)),
            pl.BlockSpec((1, H), lambda i: (0, 0)),
            pl.BlockSpec((H, D), lambda i: (0, 0)),
            pl.BlockSpec((1, D), lambda i: (0, 0)),
        ],
        out_specs=pl.BlockSpec((2, _NB, DH), lambda i: (0, i, 0)),
        out_shape=jax.ShapeDtypeStruct((2, N, DH), jnp.float32),
    )(epsv, h_split, agg_split, W1f, b1f, W2f, b2f)


def kernel(x, edge_index, edge_attr, batch, emb_table, edgeW, edgeb,
           W1, b1, g1, be1, W2, b2, eps, g2, be2):
    del x, batch
    inv = 1.0 / jnp.sqrt(1.0 + 1e-5)
    s1 = (g1 * inv)
    W1f = W1 * s1[:, None, :]
    b1f = b1 * s1 + be1
    s2 = (g2 * inv)
    W2f = W2 * s2[:, None, :]
    b2f = b2 * s2 + be2
    epsv = (1.0 + eps).astype(jnp.float32)

    src = edge_index[0]
    dst = edge_index[1]
    srcx = jnp.concatenate([src, src + N])

    def _ilv(w):
        sh = w.shape[:-1]
        return (w.reshape(sh + (D // 32, 2, 16))
                .swapaxes(-1, -2).reshape(sh + (D,)))
    edgeWt = _ilv(edgeW).transpose(1, 0, 2).reshape(EDGE_DIM, L * D)
    edgebt = _ilv(edgeb).reshape(1, L * D)
    emb_all = _edge_emb_all(edge_attr, edgeWt, edgebt)
    emb_flat = jax.lax.bitcast_convert_type(
        emb_all.reshape(L, 2, E, DH // 2, 2),
        jnp.int32).reshape(L * E, DH)

    h_flat = jnp.broadcast_to(emb_table[0].reshape(2, 1, DH),
                              (2, N, DH)).reshape(2 * N, DH)

    for l in range(L):
        agg_flat = _sc_msg_fn(l)(h_flat, srcx, dst, emb_flat)
        h_split = _mlp(epsv[l].reshape(1), h_flat.reshape(2, N, DH),
                       agg_flat.reshape(2, N, DH),
                       W1f[l], b1f[l].reshape(1, H), W2f[l],
                       b2f[l].reshape(1, D),
                       relu_out=(l != L - 1))
        h_flat = h_split.reshape(2 * N, DH)

    h_split = h_flat.reshape(2, N, DH)
    return jnp.concatenate([h_split[0], h_split[1]], axis=1)


# speedup vs baseline: 3.6770x; 3.6770x over previous
import functools

import jax
import jax.numpy as jnp
from jax import lax
from jax.experimental import pallas as pl
from jax.experimental.pallas import tpu as pltpu
from jax.experimental.pallas import tpu_sc as plsc

L = 5
N = 10000
E = 160000
D = 256
H = 512
EDGE_DIM = 7

DH = D // 2
NT = 16
EPT = E // NT
K = 40
NCHUNK = EPT // K
ROWS_PT = 640
ROWS_LAST = N - 15 * ROWS_PT


NB = 3
CPT = EPT // K


def _sc_msg_body(layer, h_hbm, src_hbm, dst_hbm, emb_hbm, out_hbm,
                 src0, src1, src2, dst0, dst1, dst2, rows0, rows1, rows2,
                 emb0, emb1, emb2, agg_sh,
                 g0, g1, g2, m0, m1, m2, s0, s1, s2, d0, d1, d2, c0, c1, c2):
    c = lax.axis_index("c")
    s = lax.axis_index("s")
    rows = [rows0, rows1, rows2]
    embs = [emb0, emb1, emb2]
    srcs = [src0, src1, src2]
    dsts = [dst0, dst1, dst2]
    gsem = [g0, g1, g2]
    msem = [m0, m1, m2]
    ssem = [s0, s1, s2]
    dsem = [d0, d1, d2]
    csem = [c0, c1, c2]

    e0 = s * EPT
    erow0 = (2 * layer * E + c * E + e0) // K

    def _zero_row(e, carry):
        for j in range(DH // 16):
            rows0[e, pl.ds(j * 16, 16)] = jnp.zeros((16,), jnp.float32)
        return carry
    lax.fori_loop(0, K, _zero_row, 0)

    r0 = s * ROWS_PT
    nz = jnp.where(s == NT - 1, ROWS_LAST // K, ROWS_PT // K)

    def _zero_agg(i, carry):
        pltpu.sync_copy(rows0, agg_sh.at[pl.ds(r0 + i * K, K)])
        return carry
    lax.fori_loop(0, nz, _zero_agg, 0)

    plsc.subcore_barrier()

    def _issue_src(i, b):
        return pltpu.async_copy(src_hbm.at[pl.ds(c * E + e0 + i * K, K)],
                                srcs[b], csem[b])

    def _wait_src(i, b):
        pltpu.make_async_copy(src_hbm.at[pl.ds(c * E + e0 + i * K, K)],
                              srcs[b], csem[b]).wait()

    def _issue_gather(b):
        return pltpu.async_copy(h_hbm.at[srcs[b]], rows[b], gsem[b])

    def _issue_emb(i, b):
        return pltpu.async_copy(emb_hbm.at[pl.ds((erow0 + i) * K, K)],
                                embs[b], msem[b])

    def _issue_dst(i, b):
        return pltpu.async_copy(dst_hbm.at[pl.ds(e0 + i * K, K)],
                                dsts[b], dsem[b])

    def _issue_scatter(b):
        return pltpu.async_copy(rows[b], agg_sh.at[dsts[b]], ssem[b],
                                add=True)

    def _wait_gather(b):
        pltpu.make_async_copy(h_hbm.at[srcs[b]], rows[b], gsem[b]).wait()

    def _wait_emb(i, b):
        pltpu.make_async_copy(emb_hbm.at[pl.ds((erow0 + i) * K, K)],
                              embs[b], msem[b]).wait()

    def _wait_dst(i, b):
        pltpu.make_async_copy(dst_hbm.at[pl.ds(e0 + i * K, K)],
                              dsts[b], dsem[b]).wait()

    def _wait_scatter(b):
        pltpu.make_async_copy(rows[b], agg_sh.at[dsts[b]], ssem[b]).wait()

    def _compute(b):
        rv = rows[b]
        ev = embs[b]

        def _edge(e2, cc):
            for u2 in range(2):
                e = e2 * 2 + u2
                for j in range(DH // 16):
                    sl = pl.ds(j * 16, 16)
                    rv[e, sl] = jnp.maximum(rv[e, sl] + ev[e, sl], 0.0)
            return cc
        lax.fori_loop(0, K // 2, _edge, 0)

    _issue_src(0, 0)
    _issue_src(1, 1)
    _issue_src(2, 2)
    _issue_emb(0, 0)
    _issue_emb(1, 1)
    _issue_emb(2, 2)
    _issue_dst(0, 0)
    _issue_dst(1, 1)
    _wait_src(0, 0)
    _issue_gather(0)
    _wait_src(1, 1)
    _issue_gather(1)

    def _step(i, b, g=None):
        tail = g is None
        b2 = (b + 2) % NB
        _wait_gather(b)
        if tail:
            if i >= 1:
                _wait_scatter(b2)
        elif b == 0:
            @pl.when(g >= 1)
            def _():
                _wait_scatter(b2)
        else:
            _wait_scatter(b2)
        if (not tail) or i + 2 <= CPT - 1:
            _wait_src(i + 2, b2)
            _issue_gather(b2)
            _issue_dst(i + 2, b2)
        if (not tail) or i + 3 <= CPT - 1:
            _issue_src(i + 3, b)
        _wait_emb(i, b)
        _compute(b)
        _wait_dst(i, b)
        if (not tail) or i < CPT - 1:
            _issue_scatter(b)
        else:
            pltpu.sync_copy(rows[b], agg_sh.at[dsts[b]], add=True)
        if (not tail) or i + 3 <= CPT - 1:
            _issue_emb(i + 3, b)

    NMAIN = ((CPT - 5) // NB) * NB

    def _main(g, carry):
        for b in range(NB):
            _step(g * NB + b, b, g=g)
        return carry
    lax.fori_loop(0, NMAIN // NB, _main, 0)

    for i in range(NMAIN, CPT):
        _step(i, i % NB)

    plsc.subcore_barrier()

    def _wb(i, carry):
        pltpu.sync_copy(agg_sh.at[pl.ds(r0 + i * K, K)], rows0)
        pltpu.sync_copy(rows0, out_hbm.at[pl.ds(c * N + r0 + i * K, K)])
        return carry
    lax.fori_loop(0, nz, _wb, 0)


@functools.cache
def _sc_msg_fn(layer):
    return pl.kernel(
        functools.partial(_sc_msg_body, layer),
        out_type=jax.ShapeDtypeStruct((2 * N, DH), jnp.float32),
        mesh=plsc.VectorSubcoreMesh(core_axis_name="c", subcore_axis_name="s"),
        scratch_types=(
            [pltpu.VMEM((K,), jnp.int32) for _ in range(2 * NB)]
            + [pltpu.VMEM((K, DH), jnp.float32) for _ in range(2 * NB)]
            + [pltpu.VMEM_SHARED((N, DH), jnp.float32)]
            + [pltpu.SemaphoreType.DMA for _ in range(5 * NB)]
        ),
    )


def _edge_emb_kernel(attr_ref, Wt_ref, bt_ref, out_ref):
    emb = jnp.dot(attr_ref[...], Wt_ref[...], preferred_element_type=jnp.float32)
    emb = emb + bt_ref[...]
    for l in range(L):
        out_ref[l, 0] = emb[:, l * D:l * D + DH]
        out_ref[l, 1] = emb[:, l * D + DH:(l + 1) * D]


def _mlp_kernel(eps_ref, h_ref, agg_ref, W1_ref, b1_ref, W2_ref, b2_ref,
                out_ref, *, relu_out):
    h = jnp.concatenate([h_ref[0], h_ref[1]], axis=1)
    a = jnp.concatenate([agg_ref[0], agg_ref[1]], axis=1)
    pre = eps_ref[0] * h + a
    mid = jnp.dot(pre, W1_ref[...], preferred_element_type=jnp.float32)
    mid = jnp.maximum(mid + b1_ref[...], 0.0)
    out = jnp.dot(mid, W2_ref[...], preferred_element_type=jnp.float32)
    out = out + b2_ref[...]
    if relu_out:
        out = jnp.maximum(out, 0.0)
    out_ref[0] = out[:, :DH]
    out_ref[1] = out[:, DH:]


_EEB = 1000
_NB = 1000


def _edge_emb_all(edge_attr, edgeWt, edgebt):
    return pl.pallas_call(
        _edge_emb_kernel,
        grid=(E // _EEB,),
        in_specs=[
            pl.BlockSpec((_EEB, EDGE_DIM), lambda e: (e, 0)),
            pl.BlockSpec((EDGE_DIM, L * D), lambda e: (0, 0)),
            pl.BlockSpec((1, L * D), lambda e: (0, 0)),
        ],
        out_specs=pl.BlockSpec((L, 2, _EEB, DH), lambda e: (0, 0, e, 0)),
        out_shape=jax.ShapeDtypeStruct((L, 2, E, DH), jnp.float32),
    )(edge_attr, edgeWt, edgebt)


def _mlp(epsv, h_split, agg_split, W1f, b1f, W2f, b2f, relu_out):
    return pl.pallas_call(
        functools.partial(_mlp_kernel, relu_out=relu_out),
        grid=(N // _NB,),
        in_specs=[
            pl.BlockSpec(memory_space=pltpu.SMEM),
            pl.BlockSpec((2, _NB, DH), lambda i: (0, i, 0)),
            pl.BlockSpec((2, _NB, DH), lambda i: (0, i, 0)),
            pl.BlockSpec((D, H), lambda i: (0, 0)),
            pl.BlockSpec((1, H), lambda i: (0, 0)),
            pl.BlockSpec((H, D), lambda i: (0, 0)),
            pl.BlockSpec((1, D), lambda i: (0, 0)),
        ],
        out_specs=pl.BlockSpec((2, _NB, DH), lambda i: (0, i, 0)),
        out_shape=jax.ShapeDtypeStruct((2, N, DH), jnp.float32),
    )(epsv, h_split, agg_split, W1f, b1f, W2f, b2f)


def kernel(x, edge_index, edge_attr, batch, emb_table, edgeW, edgeb,
           W1, b1, g1, be1, W2, b2, eps, g2, be2):
    del x, batch
    inv = 1.0 / jnp.sqrt(1.0 + 1e-5)
    s1 = (g1 * inv)
    W1f = W1 * s1[:, None, :]
    b1f = b1 * s1 + be1
    s2 = (g2 * inv)
    W2f = W2 * s2[:, None, :]
    b2f = b2 * s2 + be2
    epsv = (1.0 + eps).astype(jnp.float32)

    src = edge_index[0]
    dst = edge_index[1]
    srcx = jnp.concatenate([src, src + N])

    edgeWt = edgeW.transpose(1, 0, 2).reshape(EDGE_DIM, L * D)
    edgebt = edgeb.reshape(1, L * D)
    emb_all = _edge_emb_all(edge_attr, edgeWt, edgebt)
    emb_flat = emb_all.reshape(L * 2 * E, DH)

    h_flat = jnp.broadcast_to(emb_table[0].reshape(2, 1, DH),
                              (2, N, DH)).reshape(2 * N, DH)

    for l in range(L):
        agg_flat = _sc_msg_fn(l)(h_flat, srcx, dst, emb_flat)
        h_split = _mlp(epsv[l].reshape(1), h_flat.reshape(2, N, DH),
                       agg_flat.reshape(2, N, DH),
                       W1f[l], b1f[l].reshape(1, H), W2f[l],
                       b2f[l].reshape(1, D),
                       relu_out=(l != L - 1))
        h_flat = h_split.reshape(2 * N, DH)

    h_split = h_flat.reshape(2, N, DH)
    return jnp.concatenate([h_split[0], h_split[1]], axis=1)
